# baseline (speedup 1.0000x reference)
import jax
import jax.numpy as jnp
from jax import lax
from jax.experimental import pallas as pl
from jax.experimental.pallas import tpu as pltpu
from jax.experimental.pallas import tpu_sc as plsc

L = 16


def _bmf_body(uid_hbm, iid_hbm, utab_hbm, itab_hbm, ub_hbm, ib_hbm, gb_hbm,
              out_hbm,
              uidx_v, iidx_v, urows_v, irows_v, ub_v, ib_v, out_v, gb_v,
              sem0, sem1, sem2, sem3):
    nc = plsc.get_sparse_core_info().num_cores
    wid = lax.axis_index("s") * nc + lax.axis_index("c")
    base = wid * uidx_v.shape[0]

    pltpu.sync_copy(uid_hbm.at[pl.ds(base, uidx_v.shape[0])], uidx_v)
    pltpu.sync_copy(iid_hbm.at[pl.ds(base, iidx_v.shape[0])], iidx_v)
    pltpu.sync_copy(gb_hbm, gb_v)

    bpw = uidx_v.shape[0]
    d_dim = utab_hbm.shape[1]

    h0 = pltpu.async_copy(utab_hbm.at[uidx_v], urows_v, sem0)
    h1 = pltpu.async_copy(itab_hbm.at[iidx_v], irows_v, sem1)
    h2 = pltpu.async_copy(ub_hbm.at[uidx_v], ub_v, sem2)
    h3 = pltpu.async_copy(ib_hbm.at[iidx_v], ib_v, sem3)
    h0.wait()
    h1.wait()
    h2.wait()
    h3.wait()

    gb = gb_v[...]
    iota = lax.iota(jnp.int32, L)

    def body(g, carry):
        r0 = g * L
        rows = r0 + iota
        acc = ub_v[pl.ds(r0, L)] + ib_v[pl.ds(r0, L)] + gb
        for d in range(d_dim):
            col = jnp.full((L,), d, jnp.int32)
            u = plsc.load_gather(urows_v, [rows, col])
            v = plsc.load_gather(irows_v, [rows, col])
            acc = acc + u * v
        out_v[pl.ds(r0, L)] = 1.0 / (1.0 + jnp.exp(-acc))
        return carry

    lax.fori_loop(0, bpw // L, body, 0)
    pltpu.sync_copy(out_v, out_hbm.at[pl.ds(base, bpw)])


def kernel(user_ids, item_ids, user_table, item_table, user_bias, item_bias,
           global_bias):
    batch = user_ids.shape[0]
    d_dim = user_table.shape[1]
    info = plsc.get_sparse_core_info()
    nw = info.num_cores * info.num_subcores
    bpw = batch // nw

    mesh = plsc.VectorSubcoreMesh(core_axis_name="c", subcore_axis_name="s")
    run = pl.kernel(
        _bmf_body,
        mesh=mesh,
        out_type=jax.ShapeDtypeStruct((batch,), jnp.float32),
        scratch_types=[
            pltpu.VMEM((bpw,), jnp.int32),
            pltpu.VMEM((bpw,), jnp.int32),
            pltpu.VMEM((bpw, d_dim), jnp.float32),
            pltpu.VMEM((bpw, d_dim), jnp.float32),
            pltpu.VMEM((bpw,), jnp.float32),
            pltpu.VMEM((bpw,), jnp.float32),
            pltpu.VMEM((bpw,), jnp.float32),
            pltpu.VMEM((L,), jnp.float32),
            pltpu.SemaphoreType.DMA,
            pltpu.SemaphoreType.DMA,
            pltpu.SemaphoreType.DMA,
            pltpu.SemaphoreType.DMA,
        ],
        compiler_params=pltpu.CompilerParams(
            needs_layout_passes=False, use_tc_tiling_on_sc=False),
    )
    out = run(user_ids.astype(jnp.int32), item_ids.astype(jnp.int32),
              user_table, item_table,
              user_bias.reshape(-1), item_bias.reshape(-1),
              jnp.broadcast_to(global_bias, (L,)))
    return out.reshape(batch, 1)
